# baseline (speedup 1.0000x reference)
import functools
import math

import jax
import jax.numpy as jnp
from jax import lax
from jax.experimental import pallas as pl
from jax.experimental.pallas import tpu as pltpu
from jax.experimental.pallas import tpu_sc as plsc

V = 32000
SMOOTH = 0.1
CONF = 1.0 - SMOOTH
SV = SMOOTH / (V - 1)
K_ROW = CONF * math.log(CONF) + (V - 2) * SV * math.log(SV)

BR = 512
BV = 6400

_SC_INFO = plsc.get_sparse_core_info()
_NC = _SC_INFO.num_cores
_NS = _SC_INFO.num_subcores
_L = _SC_INFO.num_lanes
_NW = _NC * _NS


def _make_sc_gather(n_rows):
    rows_per_w = n_rows // _NW
    mesh = plsc.VectorSubcoreMesh(core_axis_name="c", subcore_axis_name="s")

    @functools.partial(
        pl.kernel,
        mesh=mesh,
        out_type=jax.ShapeDtypeStruct((n_rows,), jnp.float32),
        scratch_types=[
            pltpu.VMEM((rows_per_w,), jnp.int32),
            pltpu.VMEM((rows_per_w,), jnp.int32),
            pltpu.VMEM((rows_per_w,), jnp.float32),
            pltpu.SemaphoreType.DMA,
        ],
    )
    def sc_gather(t_hbm, xflat_hbm, out_hbm, t_v, idx_v, g_v, sem):
        wid = lax.axis_index("s") * _NC + lax.axis_index("c")
        base = wid * rows_per_w
        pltpu.sync_copy(t_hbm.at[pl.ds(base, rows_per_w)], t_v)
        for j in range(rows_per_w // _L):
            tv = t_v[pl.ds(j * _L, _L)]
            row = base + j * _L + lax.iota(jnp.int32, _L)
            idx_v[pl.ds(j * _L, _L)] = row * V + tv
        pltpu.async_copy(xflat_hbm.at[idx_v], g_v, sem).wait()
        pltpu.sync_copy(g_v, out_hbm.at[pl.ds(base, rows_per_w)])

    return sc_gather


def _loss_body(t_ref, g_ref, x_ref, o_ref):
    ri = pl.program_id(0)
    vi = pl.program_id(1)

    @pl.when((ri == 0) & (vi == 0))
    def _():
        o_ref[0, 0] = 0.0

    x = x_ref[...]
    t2 = t_ref[0]
    maskf = (t2 != 0).astype(jnp.float32)
    rs = jnp.sum(x, axis=1, keepdims=True)
    partial = -SV * jnp.sum(rs * maskf)

    @pl.when(vi == 0)
    def _():
        g2 = g_ref[0]
        lp0 = x[:, 0:1]
        o_ref[0, 0] += jnp.sum(
            maskf * (K_ROW + SV * lp0 - (CONF - SV) * g2))

    o_ref[0, 0] += partial


def _loss(x2d, t3d, g3d):
    nr = x2d.shape[0] // BR
    nv = V // BV
    out = pl.pallas_call(
        _loss_body,
        grid=(nr, nv),
        in_specs=[
            pl.BlockSpec((1, BR, 1), lambda ri, vi: (ri, 0, 0)),
            pl.BlockSpec((1, BR, 1), lambda ri, vi: (ri, 0, 0)),
            pl.BlockSpec((BR, BV), lambda ri, vi: (ri, vi)),
        ],
        out_specs=pl.BlockSpec(
            (1, 1), lambda ri, vi: (0, 0), memory_space=pltpu.SMEM
        ),
        out_shape=jax.ShapeDtypeStruct((1, 1), jnp.float32),
    )(t3d, g3d, x2d)
    return out[0, 0]


def kernel(trg_tokens_logprobas, target_token_idxs):
    B, S, Vv = trg_tokens_logprobas.shape
    n_rows = B * S
    x2d = trg_tokens_logprobas.reshape(n_rows, Vv)
    t = target_token_idxs.astype(jnp.int32).reshape(n_rows)
    g = _make_sc_gather(n_rows)(t, x2d.reshape(-1))
    nr = n_rows // BR
    t3d = t.reshape(nr, BR, 1)
    g3d = g.reshape(nr, BR, 1)
    return _loss(x2d, t3d, g3d)
